# Initial kernel scaffold; baseline (speedup 1.0000x reference)
import jax
import jax.numpy as jnp
from jax.experimental import pallas as pl


def kernel(x, y, memory_x, memory_y, eye):
    raise NotImplementedError("write your pallas kernel here")


# speedup vs baseline: 2.4101x; 2.4101x over previous
import functools

import jax
import jax.numpy as jnp
from jax import lax
from jax.experimental import pallas as pl
from jax.experimental.pallas import tpu as pltpu
from jax.experimental.pallas import tpu_sc as plsc

NUM_CLASSES = 10
K = 5
N_SAMP = 10000
N_PAD = 10240
AUG_D = 32
QB = 128
N_QUERIES = 8192


_NW = 32
_B_PER_W = N_PAD // _NW
_CHUNK = 64
_NCHUNK = _B_PER_W // _CHUNK


def _sc_gather_body(table_hbm, idx_hbm, out_hbm, idx_v, rows_v, sem):
    wid = lax.axis_index("s") * 2 + lax.axis_index("c")
    base = pl.multiple_of(wid * _B_PER_W, _B_PER_W)
    pltpu.sync_copy(idx_hbm.at[wid], idx_v)
    cps = []
    for t in range(_NCHUNK):
        cps.append(pltpu.async_copy(
            table_hbm.at[idx_v.at[t]],
            rows_v.at[pl.ds(t * _CHUNK, _CHUNK)], sem))
    for cp in cps:
        cp.wait()
    pltpu.sync_copy(rows_v, out_hbm.at[pl.ds(base, _B_PER_W)])


def _sc_gather(aug, idx):
    call = functools.partial(
        pl.kernel,
        mesh=plsc.VectorSubcoreMesh(core_axis_name="c", subcore_axis_name="s"),
        out_type=jax.ShapeDtypeStruct((N_PAD, AUG_D), jnp.float32),
        scratch_types=[
            pltpu.VMEM((_NCHUNK, _CHUNK), jnp.int32),
            pltpu.VMEM((_B_PER_W, AUG_D), jnp.float32),
            pltpu.SemaphoreType.DMA,
        ],
        compiler_params=pltpu.CompilerParams(use_tc_tiling_on_sc=False),
    )(_sc_gather_body)
    return call(aug, idx)


def _vote_body(xn_ref, xf_ref, yn_ref, memT_ref, lab_ref, out_ref):
    mm = jnp.dot(xf_ref[:, :], memT_ref[:, :],
                 preferred_element_type=jnp.float32)
    d = (xn_ref[:, :] + yn_ref[:, :]) - 2.0 * mm
    col = lax.broadcasted_iota(jnp.int32, (QB, N_PAD), 1)
    d = jnp.where(col < N_SAMP, d, jnp.inf)
    sel_total = jnp.zeros((QB, N_PAD), jnp.float32)
    for _ in range(K):
        m = jnp.min(d, axis=1, keepdims=True)
        idx = jnp.min(jnp.where(d == m, col, N_PAD), axis=1, keepdims=True)
        sel = col == idx
        sel_total = sel_total + sel.astype(jnp.float32)
        d = jnp.where(sel, jnp.inf, d)
    lab_oh = (lab_ref[:, :] ==
              lax.broadcasted_iota(jnp.int32, (N_PAD, NUM_CLASSES), 1
                                   ).astype(jnp.float32)).astype(jnp.float32)
    counts = jnp.dot(sel_total, lab_oh, preferred_element_type=jnp.float32)
    best = jnp.max(counts, axis=1, keepdims=True)
    cls = lax.broadcasted_iota(jnp.int32, (QB, NUM_CLASSES), 1)
    pred = jnp.min(jnp.where(counts == best, cls, NUM_CLASSES), axis=1,
                   keepdims=True)
    out_ref[:, :] = (cls == pred).astype(jnp.float32)


def _vote_call(xn, xf, yn, memT, lab):
    grid = N_QUERIES // QB
    return pl.pallas_call(
        _vote_body,
        grid=(grid,),
        in_specs=[
            pl.BlockSpec((QB, 1), lambda i: (i, 0)),
            pl.BlockSpec((QB, 16), lambda i: (i, 0)),
            pl.BlockSpec((1, N_PAD), lambda i: (0, 0)),
            pl.BlockSpec((16, N_PAD), lambda i: (0, 0)),
            pl.BlockSpec((N_PAD, 1), lambda i: (0, 0)),
        ],
        out_specs=pl.BlockSpec((QB, NUM_CLASSES), lambda i: (i, 0)),
        out_shape=jax.ShapeDtypeStruct((N_QUERIES, NUM_CLASSES), jnp.float32),
    )(xn, xf, yn, memT, lab)


def kernel(x, y, memory_x, memory_y, eye):
    b, c, h, w = x.shape
    xf = jnp.transpose(x, (0, 2, 3, 1)).reshape(b * h * w, c)
    n = xf.shape[0]
    mem_idx = jax.random.randint(jax.random.key(1234), (N_SAMP,), 0, n,
                                 dtype=jnp.int32)
    idx_pad = jnp.concatenate(
        [mem_idx, jnp.zeros((N_PAD - N_SAMP,), jnp.int32)]).reshape(
            _NW, _NCHUNK, _CHUNK)
    aug = jnp.concatenate(
        [memory_x, memory_y.astype(jnp.float32),
         jnp.zeros((memory_x.shape[0], AUG_D - 17), jnp.float32)], axis=1)
    sampled = _sc_gather(aug, idx_pad)
    mem_s = sampled[:, :16]
    lab = sampled[:, 16:17]
    xn = jnp.sum(xf ** 2, axis=1).reshape(-1, 1)
    yn = jnp.sum(mem_s ** 2, axis=1).reshape(1, -1)
    one_hot = _vote_call(xn, xf, yn, mem_s.T, lab)
    return jnp.transpose(one_hot.reshape(b, h, w, NUM_CLASSES), (0, 3, 1, 2))


# speedup vs baseline: 4.0753x; 1.6909x over previous
import functools

import jax
import jax.numpy as jnp
from jax import lax
from jax.experimental import pallas as pl
from jax.experimental.pallas import tpu as pltpu
from jax.experimental.pallas import tpu_sc as plsc

NUM_CLASSES = 10
K = 5
N_SAMP = 10000
N_PAD = 10240
AUG_D = 32
QB = 512
SW = 128
NCH = N_PAD // SW
N_QUERIES = 8192


_NW = 32
_B_PER_W = N_PAD // _NW
_CHUNK = 64
_NCHUNK = _B_PER_W // _CHUNK


def _sc_gather_body(table_hbm, idx_hbm, out_hbm, idx_v, rows_v, sem):
    wid = lax.axis_index("s") * 2 + lax.axis_index("c")
    base = pl.multiple_of(wid * _B_PER_W, _B_PER_W)
    pltpu.sync_copy(idx_hbm.at[wid], idx_v)
    cps = []
    for t in range(_NCHUNK):
        cps.append(pltpu.async_copy(
            table_hbm.at[idx_v.at[t]],
            rows_v.at[pl.ds(t * _CHUNK, _CHUNK)], sem))
    for cp in cps:
        cp.wait()
    pltpu.sync_copy(rows_v, out_hbm.at[pl.ds(base, _B_PER_W)])


def _sc_gather(aug, idx):
    call = functools.partial(
        pl.kernel,
        mesh=plsc.VectorSubcoreMesh(core_axis_name="c", subcore_axis_name="s"),
        out_type=jax.ShapeDtypeStruct((N_PAD, AUG_D), jnp.float32),
        scratch_types=[
            pltpu.VMEM((_NCHUNK, _CHUNK), jnp.int32),
            pltpu.VMEM((_B_PER_W, AUG_D), jnp.float32),
            pltpu.SemaphoreType.DMA,
        ],
        compiler_params=pltpu.CompilerParams(use_tc_tiling_on_sc=False),
    )(_sc_gather_body)
    return call(aug, idx)


_BIGC = 2 ** 30


def _vote_body(xn_ref, xf_ref, yn_ref, memT_ref, code_ref, out_ref):
    xb = xf_ref[:, :]
    xn = xn_ref[:, :]
    inf = jnp.float32(jnp.inf)
    T = [jnp.full((QB, SW), inf, jnp.float32) for _ in range(K)]
    C = [jnp.full((QB, SW), _BIGC, jnp.int32) for _ in range(K)]
    for s in range(NCH):
        lo = s * SW
        mm = jnp.dot(xb, memT_ref[:, lo:lo + SW],
                     preferred_element_type=jnp.float32)
        X = (xn + yn_ref[:, lo:lo + SW]) + mm
        J = code_ref[:, lo:lo + SW]
        c = [X < T[k] for k in range(K)]
        newT = [jnp.where(c[0], X, T[0])]
        newC = [jnp.where(c[0], J, C[0])]
        for k in range(1, K):
            newT.append(jnp.where(c[k], jnp.where(c[k - 1], T[k - 1], X),
                                  T[k]))
            newC.append(jnp.where(c[k], jnp.where(c[k - 1], C[k - 1], J),
                                  C[k]))
        T, C = newT, newC
    V = jnp.concatenate(T, axis=1)
    Cc = jnp.concatenate(C, axis=1)
    cls = lax.broadcasted_iota(jnp.int32, (QB, NUM_CLASSES), 1)
    counts = jnp.zeros((QB, NUM_CLASSES), jnp.float32)
    for _ in range(K):
        m = jnp.min(V, axis=1, keepdims=True)
        cm = jnp.min(jnp.where(V == m, Cc, _BIGC), axis=1, keepdims=True)
        lab_k = jnp.bitwise_and(cm, 15)
        counts = counts + (cls == lab_k).astype(jnp.float32)
        V = jnp.where(Cc == cm, inf, V)
    best = jnp.max(counts, axis=1, keepdims=True)
    pred = jnp.min(jnp.where(counts == best, cls, NUM_CLASSES), axis=1,
                   keepdims=True)
    out_ref[:, :] = (cls == pred).astype(jnp.float32)


def _vote_call(xn, xf, yn, memT, code):
    grid = N_QUERIES // QB
    return pl.pallas_call(
        _vote_body,
        grid=(grid,),
        in_specs=[
            pl.BlockSpec((QB, 1), lambda i: (i, 0)),
            pl.BlockSpec((QB, 16), lambda i: (i, 0)),
            pl.BlockSpec((1, N_PAD), lambda i: (0, 0)),
            pl.BlockSpec((16, N_PAD), lambda i: (0, 0)),
            pl.BlockSpec((1, N_PAD), lambda i: (0, 0)),
        ],
        out_specs=pl.BlockSpec((QB, NUM_CLASSES), lambda i: (i, 0)),
        out_shape=jax.ShapeDtypeStruct((N_QUERIES, NUM_CLASSES), jnp.float32),
    )(xn, xf, yn, memT, code)


def kernel(x, y, memory_x, memory_y, eye):
    b, c, h, w = x.shape
    xf = jnp.transpose(x, (0, 2, 3, 1)).reshape(b * h * w, c)
    n = xf.shape[0]
    mem_idx = jax.random.randint(jax.random.key(1234), (N_SAMP,), 0, n,
                                 dtype=jnp.int32)
    idx_pad = jnp.concatenate(
        [mem_idx, jnp.zeros((N_PAD - N_SAMP,), jnp.int32)]).reshape(
            _NW, _NCHUNK, _CHUNK)
    aug = jnp.concatenate(
        [memory_x, memory_y.astype(jnp.float32),
         jnp.zeros((memory_x.shape[0], AUG_D - 17), jnp.float32)], axis=1)
    sampled = _sc_gather(aug, idx_pad)
    mem_s = sampled[:, :16]
    col = jnp.arange(N_PAD, dtype=jnp.int32)
    code = (col * 16 + sampled[:, 16].astype(jnp.int32)).reshape(1, -1)
    xn = jnp.sum(xf ** 2, axis=1).reshape(-1, 1)
    yn = jnp.sum(mem_s ** 2, axis=1)
    yn = jnp.where(col < N_SAMP, yn, jnp.inf).reshape(1, -1)
    memT = mem_s.T * jnp.float32(-2.0)
    one_hot = _vote_call(xn, xf, yn, memT, code)
    return jnp.transpose(one_hot.reshape(b, h, w, NUM_CLASSES), (0, 3, 1, 2))
